# baseline (speedup 1.0000x reference)
import jax
import jax.numpy as jnp
from jax import lax
from jax.experimental import pallas as pl
from jax.experimental.pallas import tpu as pltpu

NBUF = 8
CHUNK = 8


def _body(lp_hbm, lt_hbm, x_ref, y_ref, out_ref,
          lpb, ltb, pos_ref, acc_ref, sems):
    a = y_ref.shape[1]
    nchunks = lp_hbm.shape[0] // CHUNK

    def start(c, slot):
        pltpu.make_async_copy(
            lp_hbm.at[pl.ds(c * CHUNK, CHUNK), :], lpb.at[slot], sems.at[slot, 0]
        ).start()
        pltpu.make_async_copy(
            lt_hbm.at[pl.ds(c * CHUNK, CHUNK), :], ltb.at[slot], sems.at[slot, 1]
        ).start()

    def wait(c, slot):
        pltpu.make_async_copy(
            lp_hbm.at[pl.ds(c * CHUNK, CHUNK), :], lpb.at[slot], sems.at[slot, 0]
        ).wait()
        pltpu.make_async_copy(
            lt_hbm.at[pl.ds(c * CHUNK, CHUNK), :], ltb.at[slot], sems.at[slot, 1]
        ).wait()

    for c in range(min(NBUF, nchunks)):
        start(c, c)

    y = y_ref[...]
    lane = lax.broadcasted_iota(jnp.int32, y.shape, 1)
    valid = lane < a
    t = (valid & (y == 1)).astype(jnp.float32)
    pos = (valid & (y > 0)).astype(jnp.float32)
    x = x_ref[...].reshape(y.shape)
    z = 2.0 * x * (2.0 * t - 1.0) + 1.0
    neg_logpt = jnp.log(1.0 + jnp.exp(-jnp.abs(z))) - jnp.minimum(z, 0.0)
    w = 0.75 - 0.5 * t
    cls_elem = jnp.where(valid & (y > -1), w * neg_logpt, 0.0)
    cls_sum = 0.5 * jnp.sum(cls_elem)
    np_sum = jnp.sum(pos)
    pos_ref[...] = pos.reshape(pos_ref.shape)

    def chunk_body(c, acc):
        slot = lax.rem(c, NBUF)
        wait(c, slot)
        d = lpb[slot] - ltb[slot]
        nxt = c + NBUF

        @pl.when(nxt < nchunks)
        def _():
            start(nxt, lax.rem(nxt, NBUF))

        ad = jnp.abs(d)
        q = jnp.minimum(ad, 1.0)
        sl1 = q * (ad - 0.5 * q)
        rs = jnp.sum(sl1.reshape(-1, 8, a), axis=1)
        lane1 = lax.broadcasted_iota(jnp.int32, rs.shape, 1)
        rs = jnp.where(lane1 < a, rs, 0.0)
        bb = rs.shape[0]
        pr = pos_ref[pl.ds(c * bb, bb), 0, :]
        return acc + jnp.sum(rs * pr)

    loc_sum = lax.fori_loop(0, nchunks, chunk_body, 0.0, unroll=False)
    acc_ref[0] = loc_sum
    inv = 1.0 / np_sum
    out_ref[0] = (0.2 * loc_sum + cls_sum) * inv


def kernel(loc_preds, loc_targets, cls_preds, cls_targets):
    b, a, dd = loc_preds.shape
    lp = jnp.transpose(loc_preds, (0, 2, 1)).reshape(b * dd, a)
    lt = jnp.transpose(loc_targets, (0, 2, 1)).reshape(b * dd, a)
    x = jnp.transpose(cls_preds, (0, 2, 1))
    y = cls_targets

    out = pl.pallas_call(
        _body,
        in_specs=[
            pl.BlockSpec(memory_space=pl.ANY),
            pl.BlockSpec(memory_space=pl.ANY),
            pl.BlockSpec((b, 1, a), lambda: (0, 0, 0)),
            pl.BlockSpec((b, a), lambda: (0, 0)),
        ],
        out_specs=pl.BlockSpec(memory_space=pltpu.SMEM),
        out_shape=jax.ShapeDtypeStruct((1,), jnp.float32),
        scratch_shapes=[
            pltpu.VMEM((NBUF, CHUNK, a), jnp.float32),
            pltpu.VMEM((NBUF, CHUNK, a), jnp.float32),
            pltpu.VMEM((b, 1, a), jnp.float32),
            pltpu.SMEM((1,), jnp.float32),
            pltpu.SemaphoreType.DMA((NBUF, 2)),
        ],
    )(lp, lt, x, y)
    return out[0]
